# baseline (speedup 1.0000x reference)
import functools

import jax
import jax.numpy as jnp
from jax import lax
from jax.experimental import pallas as pl
from jax.experimental.pallas import tpu as pltpu
from jax.experimental.pallas import tpu_sc as plsc

_B = 32
_L = 9
_NW = 32
_CH = 64


def _sc_gather(table, idx):
    M = idx.shape[0]
    R = table.shape[1]
    per_w = M // _NW
    esz = table.dtype.itemsize
    ch = next(c for c in (512, 256, 128, 64, 32, 16, 8)
              if per_w % (2 * c) == 0 and 2 * c * R * esz <= 400_000)
    nch = per_w // ch
    idx2 = idx.reshape(M // ch, ch)
    mesh = plsc.VectorSubcoreMesh(core_axis_name="c", subcore_axis_name="s")

    @functools.partial(
        pl.kernel,
        out_type=jax.ShapeDtypeStruct((M, R), table.dtype),
        mesh=mesh,
        compiler_params=pltpu.CompilerParams(use_tc_tiling_on_sc=False),
        scratch_types=[
            pltpu.VMEM((nch, ch), jnp.int32),
            pltpu.VMEM((ch, R), table.dtype),
            pltpu.VMEM((ch, R), table.dtype),
            pltpu.SemaphoreType.DMA,
            pltpu.SemaphoreType.DMA,
            pltpu.SemaphoreType.DMA,
            pltpu.SemaphoreType.DMA,
        ],
    )
    def k(tab_hbm, idx_hbm, out_hbm, idx_v, rows0, rows1, g0, g1, o0, o1):
        wid = lax.axis_index("s") * 2 + lax.axis_index("c")
        base = wid * nch
        bufs = ((rows0, g0, o0), (rows1, g1, o1))
        pltpu.sync_copy(idx_hbm.at[pl.ds(base, nch)], idx_v)
        for b, (rows, g, _) in enumerate(bufs):
            pltpu.async_copy(tab_hbm.at[idx_v.at[b]], rows, g)

        @pl.loop(0, nch - 2, step=2)
        def _(i):
            for b, (rows, g, o) in enumerate(bufs):
                pltpu.make_async_copy(tab_hbm.at[idx_v.at[i + b]], rows, g).wait()
                pltpu.async_copy(rows, out_hbm.at[pl.ds((base + i + b) * ch, ch)], o)
            for b, (rows, g, o) in enumerate(bufs):
                pltpu.make_async_copy(
                    rows, out_hbm.at[pl.ds((base + i + b) * ch, ch)], o).wait()
                pltpu.async_copy(tab_hbm.at[idx_v.at[i + 2 + b]], rows, g)

        for b, (rows, g, o) in enumerate(bufs):
            j = nch - 2 + b
            pltpu.make_async_copy(tab_hbm.at[idx_v.at[j]], rows, g).wait()
            pltpu.async_copy(rows, out_hbm.at[pl.ds((base + j) * ch, ch)], o)
        for b, (rows, g, o) in enumerate(bufs):
            j = nch - 2 + b
            pltpu.make_async_copy(
                rows, out_hbm.at[pl.ds((base + j) * ch, ch)], o).wait()

    return k(table, idx2)


def _tc_spiral_conv(g3, bd, bias_row, act):
    nl, V, rin = g3.shape
    rout = bd.shape[3]
    vb = 512 if max(rin, rout) <= 512 else 256

    def body(g_ref, w_ref, b_ref, o_ref):
        acc = jnp.zeros((vb, rout), jnp.float32)
        for l in range(nl):
            for s in range(2):
                acc = acc + lax.dot(
                    g_ref[l], w_ref[s, l], preferred_element_type=jnp.float32)
        acc = acc + b_ref[...]
        if act:
            acc = jnp.where(acc > 0, acc, jnp.exp(acc) - 1.0)
        o_ref[...] = acc.astype(jnp.bfloat16)

    return pl.pallas_call(
        body,
        grid=(V // vb,),
        in_specs=[
            pl.BlockSpec((nl, vb, rin), lambda i: (0, i, 0)),
            pl.BlockSpec((2, nl, rin, rout), lambda i: (0, 0, 0, 0)),
            pl.BlockSpec((1, rout), lambda i: (0, 0)),
        ],
        out_specs=pl.BlockSpec((vb, rout), lambda i: (i, 0)),
        out_shape=jax.ShapeDtypeStruct((V, rout), jnp.bfloat16),
    )(g3, bd, bias_row)


def _tc_fanout(h, bd):
    V, rin = h.shape
    _, nl, _, rout = bd.shape
    vb = 512 if max(rin, rout) <= 512 else 256

    def body(h_ref, w_ref, y_ref):
        hb = h_ref[...]
        for l in range(nl):
            y_ref[l] = (lax.dot(hb, w_ref[0, l], preferred_element_type=jnp.float32)
                        + lax.dot(hb, w_ref[1, l], preferred_element_type=jnp.float32)
                        ).astype(jnp.bfloat16)

    return pl.pallas_call(
        body,
        grid=(V // vb,),
        in_specs=[
            pl.BlockSpec((vb, rin), lambda i: (i, 0)),
            pl.BlockSpec((2, nl, rin, rout), lambda i: (0, 0, 0, 0)),
        ],
        out_specs=pl.BlockSpec((nl, vb, rout), lambda i: (0, i, 0)),
        out_shape=jax.ShapeDtypeStruct((nl, V, rout), jnp.bfloat16),
    )(h, bd)


def _tc_lsum(g3, bias_row, act):
    nl, V, R = g3.shape
    vb = 512

    def body(g_ref, b_ref, o_ref):
        acc = g_ref[0].astype(jnp.float32)
        for l in range(1, nl):
            acc = acc + g_ref[l].astype(jnp.float32)
        acc = acc + b_ref[...]
        if act:
            acc = jnp.where(acc > 0, acc, jnp.exp(acc) - 1.0)
        o_ref[...] = acc.astype(jnp.bfloat16)

    return pl.pallas_call(
        body,
        grid=(V // vb,),
        in_specs=[
            pl.BlockSpec((nl, vb, R), lambda i: (0, i, 0)),
            pl.BlockSpec((1, R), lambda i: (0, 0)),
        ],
        out_specs=pl.BlockSpec((vb, R), lambda i: (i, 0)),
        out_shape=jax.ShapeDtypeStruct((V, R), jnp.bfloat16),
    )(g3, bias_row)


def _tc_pool(p3, w):
    nk, V, R = p3.shape
    vb = 512

    def body(p_ref, w_ref, o_ref):
        acc = p_ref[0].astype(jnp.float32) * w_ref[:, 0:1]
        for k in range(1, nk):
            acc = acc + p_ref[k].astype(jnp.float32) * w_ref[:, k:k + 1]
        o_ref[...] = acc.astype(jnp.bfloat16)

    return pl.pallas_call(
        body,
        grid=(V // vb,),
        in_specs=[
            pl.BlockSpec((nk, vb, R), lambda i: (0, i, 0)),
            pl.BlockSpec((vb, nk), lambda i: (i, 0)),
        ],
        out_specs=pl.BlockSpec((vb, R), lambda i: (i, 0)),
        out_shape=jax.ShapeDtypeStruct((V, R), jnp.bfloat16),
    )(p3, w)


def _tc_latent(a, wen_t, ben_row, wde_t, bde_row):
    def body(a_ref, we_ref, be_ref, wd_ref, bd_ref, z_ref, d_ref):
        a = a_ref[...]
        z = (lax.dot(a, we_ref[0], preferred_element_type=jnp.float32)
             + lax.dot(a, we_ref[1], preferred_element_type=jnp.float32)
             + be_ref[...])
        z_ref[...] = z
        zb = z.astype(jnp.bfloat16)
        d_ref[...] = (lax.dot(zb, wd_ref[0], preferred_element_type=jnp.float32)
                      + lax.dot(zb, wd_ref[1], preferred_element_type=jnp.float32)
                      + bd_ref[...]).astype(jnp.bfloat16)

    return pl.pallas_call(
        body,
        out_shape=(jax.ShapeDtypeStruct((a.shape[0], wen_t.shape[2]), jnp.float32),
                   jax.ShapeDtypeStruct((a.shape[0], wde_t.shape[2]), jnp.bfloat16)),
    )(a, wen_t, ben_row, wde_t, bde_row)


def _flat(ix):
    return ix.astype(jnp.int32).T.reshape(-1)


def _flat_off(ix, n):
    f = ix.astype(jnp.int32).T
    return (f + jnp.arange(_L, dtype=jnp.int32)[:, None] * n).reshape(-1)


def _block_diag(W):
    co = W.shape[0]
    ci = W.shape[1] // _L
    wt = W.reshape(co, _L, ci).transpose(1, 2, 0)
    eye = jnp.eye(_B, dtype=W.dtype)
    bd = (eye[None, :, None, :, None] * wt[:, None, :, None, :]
          ).reshape(_L, _B * ci, _B * co)
    return _split_bf16(bd)


def _split_bf16(w):
    hi = w.astype(jnp.bfloat16)
    lo = (w - hi.astype(jnp.float32)).astype(jnp.bfloat16)
    return jnp.stack([hi, lo])


def kernel(x, W_en, b_en, W_lin_en, b_lin_en, W_lin_de, b_lin_de, W_de, b_de,
           w_d, w_u, spiral_idx, idx_d, idx_u):
    v0, c0 = x.shape[1], x.shape[2]
    h = x.transpose(1, 0, 2).reshape(v0, _B * c0).astype(jnp.bfloat16)

    for i in range(4):
        vi = spiral_idx[i].shape[0]
        g = _sc_gather(h, _flat(spiral_idx[i]))
        a = _tc_spiral_conv(g.reshape(_L, vi, -1), _block_diag(W_en[i]),
                            jnp.tile(b_en[i], _B)[None, :], act=True)
        vn = idx_d[i].shape[0]
        p = _sc_gather(a, _flat(idx_d[i]))
        h = _tc_pool(p.reshape(4, vn, -1), w_d[i])

    nv = h.shape[0]
    cl = h.shape[1] // _B
    perm1 = (jnp.arange(_B, dtype=jnp.int32)[:, None]
             + jnp.arange(nv, dtype=jnp.int32)[None, :] * _B).reshape(-1)
    a_lat = _sc_gather(h.reshape(nv * _B, cl), perm1).reshape(_B, nv * cl)
    z, dflat = _tc_latent(a_lat,
                          _split_bf16(W_lin_en.T), b_lin_en[None, :],
                          _split_bf16(W_lin_de.T), b_lin_de[None, :])
    perm2 = (jnp.arange(nv, dtype=jnp.int32)[:, None]
             + jnp.arange(_B, dtype=jnp.int32)[None, :] * nv).reshape(-1)
    d = _sc_gather(dflat.reshape(_B * nv, cl), perm2).reshape(nv, _B * cl)

    for j in range(4):
        lev = 3 - j
        vt = idx_u[lev].shape[0]
        p = _sc_gather(d, _flat(idx_u[lev]))
        d = _tc_pool(p.reshape(3, vt, -1), w_u[lev])
        bias_row = jnp.tile(b_de[j], _B)[None, :]
        if j in (0, 3):
            y = _tc_fanout(d, _block_diag(W_de[j]))
            g = _sc_gather(y.reshape(_L * vt, -1), _flat_off(spiral_idx[lev], vt))
            d = _tc_lsum(g.reshape(_L, vt, -1), bias_row, act=(j < 3))
        else:
            g = _sc_gather(d, _flat(spiral_idx[lev]))
            d = _tc_spiral_conv(g.reshape(_L, vt, -1), _block_diag(W_de[j]),
                                bias_row, act=(j < 3))

    dout = d.astype(jnp.float32).reshape(v0, _B, -1).transpose(1, 0, 2)
    return (z, dout)


# speedup vs baseline: 1.0520x; 1.0520x over previous
import functools

import jax
import jax.numpy as jnp
from jax import lax
from jax.experimental import pallas as pl
from jax.experimental.pallas import tpu as pltpu
from jax.experimental.pallas import tpu_sc as plsc

_B = 32
_L = 9
_NW = 32
_CH = 64


def _sc_gather(table, idx):
    M = idx.shape[0]
    R = table.shape[1]
    per_w = M // _NW
    esz = table.dtype.itemsize
    ch = next(c for c in (512, 256, 128, 64, 32, 16, 8)
              if per_w % (2 * c) == 0 and 2 * c * R * esz <= 400_000)
    nch = per_w // ch
    idx2 = idx.reshape(M // ch, ch)
    mesh = plsc.VectorSubcoreMesh(core_axis_name="c", subcore_axis_name="s")

    @functools.partial(
        pl.kernel,
        out_type=jax.ShapeDtypeStruct((M, R), table.dtype),
        mesh=mesh,
        compiler_params=pltpu.CompilerParams(use_tc_tiling_on_sc=False),
        scratch_types=[
            pltpu.VMEM((nch, ch), jnp.int32),
            pltpu.VMEM((ch, R), table.dtype),
            pltpu.VMEM((ch, R), table.dtype),
            pltpu.SemaphoreType.DMA,
            pltpu.SemaphoreType.DMA,
            pltpu.SemaphoreType.DMA,
            pltpu.SemaphoreType.DMA,
        ],
    )
    def k(tab_hbm, idx_hbm, out_hbm, idx_v, rows0, rows1, g0, g1, o0, o1):
        wid = lax.axis_index("s") * 2 + lax.axis_index("c")
        base = wid * nch
        bufs = ((rows0, g0, o0), (rows1, g1, o1))
        pltpu.sync_copy(idx_hbm.at[pl.ds(base, nch)], idx_v)
        for b, (rows, g, _) in enumerate(bufs):
            pltpu.async_copy(tab_hbm.at[idx_v.at[b]], rows, g)

        @pl.loop(0, nch - 2, step=2)
        def _(i):
            for b, (rows, g, o) in enumerate(bufs):
                pltpu.make_async_copy(tab_hbm.at[idx_v.at[i + b]], rows, g).wait()
                pltpu.async_copy(rows, out_hbm.at[pl.ds((base + i + b) * ch, ch)], o)
            for b, (rows, g, o) in enumerate(bufs):
                pltpu.make_async_copy(
                    rows, out_hbm.at[pl.ds((base + i + b) * ch, ch)], o).wait()
                pltpu.async_copy(tab_hbm.at[idx_v.at[i + 2 + b]], rows, g)

        for b, (rows, g, o) in enumerate(bufs):
            j = nch - 2 + b
            pltpu.make_async_copy(tab_hbm.at[idx_v.at[j]], rows, g).wait()
            pltpu.async_copy(rows, out_hbm.at[pl.ds((base + j) * ch, ch)], o)
        for b, (rows, g, o) in enumerate(bufs):
            j = nch - 2 + b
            pltpu.make_async_copy(
                rows, out_hbm.at[pl.ds((base + j) * ch, ch)], o).wait()

    return k(table, idx2)


def _tc_spiral_conv(g3, bd, bias_row, act):
    nl, V, rin = g3.shape
    rout = bd.shape[3]
    vb = 512 if max(rin, rout) <= 512 else 256

    def body(g_ref, w_ref, b_ref, o_ref):
        acc = jnp.zeros((vb, rout), jnp.float32)
        for l in range(nl):
            for s in range(1):
                acc = acc + lax.dot(
                    g_ref[l], w_ref[s, l], preferred_element_type=jnp.float32)
        acc = acc + b_ref[...]
        if act:
            acc = jnp.where(acc > 0, acc, jnp.exp(acc) - 1.0)
        o_ref[...] = acc.astype(jnp.bfloat16)

    return pl.pallas_call(
        body,
        grid=(V // vb,),
        in_specs=[
            pl.BlockSpec((nl, vb, rin), lambda i: (0, i, 0)),
            pl.BlockSpec((2, nl, rin, rout), lambda i: (0, 0, 0, 0)),
            pl.BlockSpec((1, rout), lambda i: (0, 0)),
        ],
        out_specs=pl.BlockSpec((vb, rout), lambda i: (i, 0)),
        out_shape=jax.ShapeDtypeStruct((V, rout), jnp.bfloat16),
    )(g3, bd, bias_row)


def _tc_fanout(h, bd):
    V, rin = h.shape
    _, nl, _, rout = bd.shape
    vb = 512 if max(rin, rout) <= 512 else 256

    def body(h_ref, w_ref, y_ref):
        hb = h_ref[...]
        for l in range(nl):
            y_ref[l] = lax.dot(hb, w_ref[0, l],
                               preferred_element_type=jnp.float32
                               ).astype(jnp.bfloat16)

    return pl.pallas_call(
        body,
        grid=(V // vb,),
        in_specs=[
            pl.BlockSpec((vb, rin), lambda i: (i, 0)),
            pl.BlockSpec((2, nl, rin, rout), lambda i: (0, 0, 0, 0)),
        ],
        out_specs=pl.BlockSpec((nl, vb, rout), lambda i: (0, i, 0)),
        out_shape=jax.ShapeDtypeStruct((nl, V, rout), jnp.bfloat16),
    )(h, bd)


def _tc_lsum(g3, bias_row, act):
    nl, V, R = g3.shape
    vb = 512

    def body(g_ref, b_ref, o_ref):
        acc = g_ref[0].astype(jnp.float32)
        for l in range(1, nl):
            acc = acc + g_ref[l].astype(jnp.float32)
        acc = acc + b_ref[...]
        if act:
            acc = jnp.where(acc > 0, acc, jnp.exp(acc) - 1.0)
        o_ref[...] = acc.astype(jnp.bfloat16)

    return pl.pallas_call(
        body,
        grid=(V // vb,),
        in_specs=[
            pl.BlockSpec((nl, vb, R), lambda i: (0, i, 0)),
            pl.BlockSpec((1, R), lambda i: (0, 0)),
        ],
        out_specs=pl.BlockSpec((vb, R), lambda i: (i, 0)),
        out_shape=jax.ShapeDtypeStruct((V, R), jnp.bfloat16),
    )(g3, bias_row)


def _tc_pool(p3, w):
    nk, V, R = p3.shape
    vb = 512

    def body(p_ref, w_ref, o_ref):
        acc = p_ref[0].astype(jnp.float32) * w_ref[:, 0:1]
        for k in range(1, nk):
            acc = acc + p_ref[k].astype(jnp.float32) * w_ref[:, k:k + 1]
        o_ref[...] = acc.astype(jnp.bfloat16)

    return pl.pallas_call(
        body,
        grid=(V // vb,),
        in_specs=[
            pl.BlockSpec((nk, vb, R), lambda i: (0, i, 0)),
            pl.BlockSpec((vb, nk), lambda i: (i, 0)),
        ],
        out_specs=pl.BlockSpec((vb, R), lambda i: (i, 0)),
        out_shape=jax.ShapeDtypeStruct((V, R), jnp.bfloat16),
    )(p3, w)


def _tc_latent(a, wen_t, ben_row, wde_t, bde_row):
    def body(a_ref, we_ref, be_ref, wd_ref, bd_ref, z_ref, d_ref):
        a = a_ref[...]
        z = (lax.dot(a, we_ref[0], preferred_element_type=jnp.float32)
             + be_ref[...])
        z_ref[...] = z
        zb = z.astype(jnp.bfloat16)
        d_ref[...] = (lax.dot(zb, wd_ref[0], preferred_element_type=jnp.float32)
                      + bd_ref[...]).astype(jnp.bfloat16)

    return pl.pallas_call(
        body,
        out_shape=(jax.ShapeDtypeStruct((a.shape[0], wen_t.shape[2]), jnp.float32),
                   jax.ShapeDtypeStruct((a.shape[0], wde_t.shape[2]), jnp.bfloat16)),
    )(a, wen_t, ben_row, wde_t, bde_row)


def _flat(ix):
    return ix.astype(jnp.int32).T.reshape(-1)


def _flat_off(ix, n):
    f = ix.astype(jnp.int32).T
    return (f + jnp.arange(_L, dtype=jnp.int32)[:, None] * n).reshape(-1)


def _block_diag(W):
    co = W.shape[0]
    ci = W.shape[1] // _L
    wt = W.reshape(co, _L, ci).transpose(1, 2, 0)
    eye = jnp.eye(_B, dtype=W.dtype)
    bd = (eye[None, :, None, :, None] * wt[:, None, :, None, :]
          ).reshape(_L, _B * ci, _B * co)
    return _split_bf16(bd)


def _split_bf16(w):
    hi = w.astype(jnp.bfloat16)
    lo = (w - hi.astype(jnp.float32)).astype(jnp.bfloat16)
    return jnp.stack([hi, lo])


def kernel(x, W_en, b_en, W_lin_en, b_lin_en, W_lin_de, b_lin_de, W_de, b_de,
           w_d, w_u, spiral_idx, idx_d, idx_u):
    v0, c0 = x.shape[1], x.shape[2]
    h = x.transpose(1, 0, 2).reshape(v0, _B * c0).astype(jnp.bfloat16)

    for i in range(4):
        vi = spiral_idx[i].shape[0]
        g = _sc_gather(h, _flat(spiral_idx[i]))
        a = _tc_spiral_conv(g.reshape(_L, vi, -1), _block_diag(W_en[i]),
                            jnp.tile(b_en[i], _B)[None, :], act=True)
        vn = idx_d[i].shape[0]
        p = _sc_gather(a, _flat(idx_d[i]))
        h = _tc_pool(p.reshape(4, vn, -1), w_d[i])

    nv = h.shape[0]
    cl = h.shape[1] // _B
    perm1 = (jnp.arange(_B, dtype=jnp.int32)[:, None]
             + jnp.arange(nv, dtype=jnp.int32)[None, :] * _B).reshape(-1)
    a_lat = _sc_gather(h.reshape(nv * _B, cl), perm1).reshape(_B, nv * cl)
    z, dflat = _tc_latent(a_lat,
                          _split_bf16(W_lin_en.T), b_lin_en[None, :],
                          _split_bf16(W_lin_de.T), b_lin_de[None, :])
    perm2 = (jnp.arange(nv, dtype=jnp.int32)[:, None]
             + jnp.arange(_B, dtype=jnp.int32)[None, :] * nv).reshape(-1)
    d = _sc_gather(dflat.reshape(_B * nv, cl), perm2).reshape(nv, _B * cl)

    for j in range(4):
        lev = 3 - j
        vt = idx_u[lev].shape[0]
        p = _sc_gather(d, _flat(idx_u[lev]))
        d = _tc_pool(p.reshape(3, vt, -1), w_u[lev])
        bias_row = jnp.tile(b_de[j], _B)[None, :]
        if j in (0, 3):
            y = _tc_fanout(d, _block_diag(W_de[j]))
            g = _sc_gather(y.reshape(_L * vt, -1), _flat_off(spiral_idx[lev], vt))
            d = _tc_lsum(g.reshape(_L, vt, -1), bias_row, act=(j < 3))
        else:
            g = _sc_gather(d, _flat(spiral_idx[lev]))
            d = _tc_spiral_conv(g.reshape(_L, vt, -1), _block_diag(W_de[j]),
                                bias_row, act=(j < 3))

    dout = d.astype(jnp.float32).reshape(v0, _B, -1).transpose(1, 0, 2)
    return (z, dout)


# speedup vs baseline: 1.3765x; 1.3084x over previous
import functools

import jax
import jax.numpy as jnp
from jax import lax
from jax.experimental import pallas as pl
from jax.experimental.pallas import tpu as pltpu
from jax.experimental.pallas import tpu_sc as plsc

_B = 32
_L = 9
_NW = 32
_CH = 64


def _sc_gather(table, idx):
    M = idx.shape[0]
    R = table.shape[1]
    per_w = M // _NW
    esz = table.dtype.itemsize
    ch = next(c for c in (512, 256, 128, 64, 32, 16, 8)
              if per_w % (2 * c) == 0 and 2 * c * R * esz <= 400_000)
    nch = per_w // ch
    idx2 = idx.reshape(M // ch, ch)
    mesh = plsc.VectorSubcoreMesh(core_axis_name="c", subcore_axis_name="s")

    @functools.partial(
        pl.kernel,
        out_type=jax.ShapeDtypeStruct((M, R), table.dtype),
        mesh=mesh,
        compiler_params=pltpu.CompilerParams(use_tc_tiling_on_sc=False),
        scratch_types=[
            pltpu.VMEM((nch, ch), jnp.int32),
            pltpu.VMEM((ch, R), table.dtype),
            pltpu.VMEM((ch, R), table.dtype),
            pltpu.SemaphoreType.DMA,
            pltpu.SemaphoreType.DMA,
            pltpu.SemaphoreType.DMA,
            pltpu.SemaphoreType.DMA,
        ],
    )
    def k(tab_hbm, idx_hbm, out_hbm, idx_v, rows0, rows1, g0, g1, o0, o1):
        wid = lax.axis_index("s") * 2 + lax.axis_index("c")
        base = wid * nch
        bufs = ((rows0, g0, o0), (rows1, g1, o1))
        pltpu.sync_copy(idx_hbm.at[pl.ds(base, nch)], idx_v)
        for b, (rows, g, _) in enumerate(bufs):
            pltpu.async_copy(tab_hbm.at[idx_v.at[b]], rows, g)

        @pl.loop(0, nch - 2, step=2)
        def _(i):
            for b, (rows, g, o) in enumerate(bufs):
                pltpu.make_async_copy(tab_hbm.at[idx_v.at[i + b]], rows, g).wait()
                pltpu.async_copy(rows, out_hbm.at[pl.ds((base + i + b) * ch, ch)], o)
            for b, (rows, g, o) in enumerate(bufs):
                pltpu.make_async_copy(
                    rows, out_hbm.at[pl.ds((base + i + b) * ch, ch)], o).wait()
                pltpu.async_copy(tab_hbm.at[idx_v.at[i + 2 + b]], rows, g)

        for b, (rows, g, o) in enumerate(bufs):
            j = nch - 2 + b
            pltpu.make_async_copy(tab_hbm.at[idx_v.at[j]], rows, g).wait()
            pltpu.async_copy(rows, out_hbm.at[pl.ds((base + j) * ch, ch)], o)
        for b, (rows, g, o) in enumerate(bufs):
            j = nch - 2 + b
            pltpu.make_async_copy(
                rows, out_hbm.at[pl.ds((base + j) * ch, ch)], o).wait()

    return k(table, idx2)


def _tc_spiral_conv(g3, bd, bias_row, act):
    nl, V, rin = g3.shape
    rout = bd.shape[2]
    vb = 512 if max(rin, rout) <= 512 else 256

    def body(g_ref, w_ref, b_ref, o_ref):
        acc = jnp.zeros((vb, rout), jnp.float32)
        for l in range(nl):
            acc = acc + lax.dot(
                g_ref[l].astype(jnp.bfloat16), w_ref[l],
                preferred_element_type=jnp.float32)
        acc = acc + b_ref[...]
        if act:
            acc = jnp.where(acc > 0, acc, jnp.exp(acc) - 1.0)
        o_ref[...] = acc

    return pl.pallas_call(
        body,
        grid=(V // vb,),
        in_specs=[
            pl.BlockSpec((nl, vb, rin), lambda i: (0, i, 0)),
            pl.BlockSpec((nl, rin, rout), lambda i: (0, 0, 0)),
            pl.BlockSpec((1, rout), lambda i: (0, 0)),
        ],
        out_specs=pl.BlockSpec((vb, rout), lambda i: (i, 0)),
        out_shape=jax.ShapeDtypeStruct((V, rout), jnp.float32),
    )(g3, bd, bias_row)


def _tc_fanout(h, bd):
    V, rin = h.shape
    nl, _, rout = bd.shape
    vb = 512 if max(rin, rout) <= 512 else 256

    def body(h_ref, w_ref, y_ref):
        hb = h_ref[...].astype(jnp.bfloat16)
        for l in range(nl):
            y_ref[l] = lax.dot(hb, w_ref[l], preferred_element_type=jnp.float32)

    return pl.pallas_call(
        body,
        grid=(V // vb,),
        in_specs=[
            pl.BlockSpec((vb, rin), lambda i: (i, 0)),
            pl.BlockSpec((nl, rin, rout), lambda i: (0, 0, 0)),
        ],
        out_specs=pl.BlockSpec((nl, vb, rout), lambda i: (0, i, 0)),
        out_shape=jax.ShapeDtypeStruct((nl, V, rout), jnp.float32),
    )(h, bd)


def _tc_lsum(g3, bias_row, act):
    nl, V, R = g3.shape
    vb = 512

    def body(g_ref, b_ref, o_ref):
        acc = g_ref[0]
        for l in range(1, nl):
            acc = acc + g_ref[l]
        acc = acc + b_ref[...]
        if act:
            acc = jnp.where(acc > 0, acc, jnp.exp(acc) - 1.0)
        o_ref[...] = acc

    return pl.pallas_call(
        body,
        grid=(V // vb,),
        in_specs=[
            pl.BlockSpec((nl, vb, R), lambda i: (0, i, 0)),
            pl.BlockSpec((1, R), lambda i: (0, 0)),
        ],
        out_specs=pl.BlockSpec((vb, R), lambda i: (i, 0)),
        out_shape=jax.ShapeDtypeStruct((V, R), jnp.float32),
    )(g3, bias_row)


def _tc_pool(p3, w):
    nk, V, R = p3.shape
    vb = 512

    def body(p_ref, w_ref, o_ref):
        acc = p_ref[0] * w_ref[:, 0:1]
        for k in range(1, nk):
            acc = acc + p_ref[k] * w_ref[:, k:k + 1]
        o_ref[...] = acc

    return pl.pallas_call(
        body,
        grid=(V // vb,),
        in_specs=[
            pl.BlockSpec((nk, vb, R), lambda i: (0, i, 0)),
            pl.BlockSpec((vb, nk), lambda i: (i, 0)),
        ],
        out_specs=pl.BlockSpec((vb, R), lambda i: (i, 0)),
        out_shape=jax.ShapeDtypeStruct((V, R), jnp.float32),
    )(p3, w)


def _tc_latent(a, wen_t, ben_row, wde_t, bde_row):
    def body(a_ref, we_ref, be_ref, wd_ref, bd_ref, z_ref, d_ref):
        z = lax.dot(a_ref[...].astype(jnp.bfloat16), we_ref[...],
                    preferred_element_type=jnp.float32) + be_ref[...]
        z_ref[...] = z
        d_ref[...] = lax.dot(z.astype(jnp.bfloat16), wd_ref[...],
                             preferred_element_type=jnp.float32) + bd_ref[...]

    return pl.pallas_call(
        body,
        out_shape=(jax.ShapeDtypeStruct((a.shape[0], wen_t.shape[1]), jnp.float32),
                   jax.ShapeDtypeStruct((a.shape[0], wde_t.shape[1]), jnp.float32)),
    )(a, wen_t, ben_row, wde_t, bde_row)


def _flat(ix):
    return ix.astype(jnp.int32).T.reshape(-1)


def _flat_off(ix, n):
    f = ix.astype(jnp.int32).T
    return (f + jnp.arange(_L, dtype=jnp.int32)[:, None] * n).reshape(-1)


def _block_diag(W):
    co = W.shape[0]
    ci = W.shape[1] // _L
    wt = W.reshape(co, _L, ci).transpose(1, 2, 0)
    eye = jnp.eye(_B, dtype=W.dtype)
    bd = eye[None, :, None, :, None] * wt[:, None, :, None, :]
    return bd.reshape(_L, _B * ci, _B * co).astype(jnp.bfloat16)


def kernel(x, W_en, b_en, W_lin_en, b_lin_en, W_lin_de, b_lin_de, W_de, b_de,
           w_d, w_u, spiral_idx, idx_d, idx_u):
    v0, c0 = x.shape[1], x.shape[2]
    h = x.transpose(1, 0, 2).reshape(v0, _B * c0)

    for i in range(4):
        vi = spiral_idx[i].shape[0]
        g = _sc_gather(h, _flat(spiral_idx[i]))
        a = _tc_spiral_conv(g.reshape(_L, vi, -1), _block_diag(W_en[i]),
                            jnp.tile(b_en[i], _B)[None, :], act=True)
        vn = idx_d[i].shape[0]
        p = _sc_gather(a, _flat(idx_d[i]))
        h = _tc_pool(p.reshape(4, vn, -1), w_d[i])

    nv = h.shape[0]
    cl = h.shape[1] // _B
    perm1 = (jnp.arange(_B, dtype=jnp.int32)[:, None]
             + jnp.arange(nv, dtype=jnp.int32)[None, :] * _B).reshape(-1)
    a_lat = _sc_gather(h.reshape(nv * _B, cl), perm1).reshape(_B, nv * cl)
    z, dflat = _tc_latent(a_lat,
                          W_lin_en.T.astype(jnp.bfloat16), b_lin_en[None, :],
                          W_lin_de.T.astype(jnp.bfloat16), b_lin_de[None, :])
    perm2 = (jnp.arange(nv, dtype=jnp.int32)[:, None]
             + jnp.arange(_B, dtype=jnp.int32)[None, :] * nv).reshape(-1)
    d = _sc_gather(dflat.reshape(_B * nv, cl), perm2).reshape(nv, _B * cl)

    for j in range(4):
        lev = 3 - j
        vt = idx_u[lev].shape[0]
        p = _sc_gather(d, _flat(idx_u[lev]))
        d = _tc_pool(p.reshape(3, vt, -1), w_u[lev])
        bias_row = jnp.tile(b_de[j], _B)[None, :]
        if j in (0, 3):
            y = _tc_fanout(d, _block_diag(W_de[j]))
            g = _sc_gather(y.reshape(_L * vt, -1), _flat_off(spiral_idx[lev], vt))
            d = _tc_lsum(g.reshape(_L, vt, -1), bias_row, act=(j < 3))
        else:
            g = _sc_gather(d, _flat(spiral_idx[lev]))
            d = _tc_spiral_conv(g.reshape(_L, vt, -1), _block_diag(W_de[j]),
                                bias_row, act=(j < 3))

    dout = d.reshape(v0, _B, -1).transpose(1, 0, 2)
    return (z, dout)


# speedup vs baseline: 2.0370x; 1.4799x over previous
import functools

import jax
import jax.numpy as jnp
from jax import lax
from jax.experimental import pallas as pl
from jax.experimental.pallas import tpu as pltpu
from jax.experimental.pallas import tpu_sc as plsc

_B = 32
_L = 9
_NW = 32
_CH = 64


def _sc_gather(table, idx):
    M = idx.shape[0]
    R = table.shape[1]
    per_w = M // _NW
    esz = table.dtype.itemsize
    ch = next(c for c in (512, 256, 128, 96, 64, 32, 16, 8)
              if per_w % (2 * c) == 0 and 2 * c * R * esz <= 400_000)
    nch = per_w // ch
    idx2 = idx.reshape(_NW, nch, ch)
    mesh = plsc.VectorSubcoreMesh(core_axis_name="c", subcore_axis_name="s")

    @functools.partial(
        pl.kernel,
        out_type=jax.ShapeDtypeStruct((M, R), table.dtype),
        mesh=mesh,
        compiler_params=pltpu.CompilerParams(
            use_tc_tiling_on_sc=(R * esz) % 512 == 0),
        scratch_types=[
            pltpu.VMEM((nch, ch), jnp.int32),
            pltpu.VMEM((ch, R), table.dtype),
            pltpu.VMEM((ch, R), table.dtype),
            pltpu.SemaphoreType.DMA,
            pltpu.SemaphoreType.DMA,
            pltpu.SemaphoreType.DMA,
            pltpu.SemaphoreType.DMA,
        ],
    )
    def k(tab_hbm, idx_hbm, out_hbm, idx_v, rows0, rows1, g0, g1, o0, o1):
        wid = lax.axis_index("s") * 2 + lax.axis_index("c")
        base = wid * nch
        bufs = ((rows0, g0, o0), (rows1, g1, o1))
        pltpu.sync_copy(idx_hbm.at[wid], idx_v)

        def orow(j):
            return pl.multiple_of((base + j) * ch, ch)
        for b, (rows, g, _) in enumerate(bufs):
            pltpu.async_copy(tab_hbm.at[idx_v.at[b]], rows, g)

        @pl.loop(0, nch - 2, step=2)
        def _(i):
            for b, (rows, g, o) in enumerate(bufs):
                pltpu.make_async_copy(tab_hbm.at[idx_v.at[i + b]], rows, g).wait()
                pltpu.async_copy(rows, out_hbm.at[pl.ds(orow(i + b), ch)], o)
            for b, (rows, g, o) in enumerate(bufs):
                pltpu.make_async_copy(
                    rows, out_hbm.at[pl.ds(orow(i + b), ch)], o).wait()
                pltpu.async_copy(tab_hbm.at[idx_v.at[i + 2 + b]], rows, g)

        for b, (rows, g, o) in enumerate(bufs):
            j = nch - 2 + b
            pltpu.make_async_copy(tab_hbm.at[idx_v.at[j]], rows, g).wait()
            pltpu.async_copy(rows, out_hbm.at[pl.ds(orow(j), ch)], o)
        for b, (rows, g, o) in enumerate(bufs):
            j = nch - 2 + b
            pltpu.make_async_copy(
                rows, out_hbm.at[pl.ds(orow(j), ch)], o).wait()

    return k(table, idx2)


def _tc_spiral_conv(g3, bd, bias_row, act):
    nl, V, rin = g3.shape
    rout = bd.shape[2]
    vb = 512 if max(rin, rout) <= 512 else 256

    def body(g_ref, w_ref, b_ref, o_ref):
        acc = jnp.zeros((vb, rout), jnp.float32)
        for l in range(nl):
            acc = acc + lax.dot(
                g_ref[l].astype(jnp.bfloat16), w_ref[l],
                preferred_element_type=jnp.float32)
        acc = acc + b_ref[...]
        if act:
            acc = jnp.where(acc > 0, acc, jnp.exp(acc) - 1.0)
        o_ref[...] = acc

    return pl.pallas_call(
        body,
        grid=(V // vb,),
        in_specs=[
            pl.BlockSpec((nl, vb, rin), lambda i: (0, i, 0)),
            pl.BlockSpec((nl, rin, rout), lambda i: (0, 0, 0)),
            pl.BlockSpec((1, rout), lambda i: (0, 0)),
        ],
        out_specs=pl.BlockSpec((vb, rout), lambda i: (i, 0)),
        out_shape=jax.ShapeDtypeStruct((V, rout), jnp.float32),
    )(g3, bd, bias_row)


def _tc_fanout(h, bd):
    V, rin = h.shape
    nl, _, rout = bd.shape
    vb = 512 if max(rin, rout) <= 512 else 256

    def body(h_ref, w_ref, y_ref):
        hb = h_ref[...].astype(jnp.bfloat16)
        for l in range(nl):
            y_ref[l] = lax.dot(hb, w_ref[l], preferred_element_type=jnp.float32)

    return pl.pallas_call(
        body,
        grid=(V // vb,),
        in_specs=[
            pl.BlockSpec((vb, rin), lambda i: (i, 0)),
            pl.BlockSpec((nl, rin, rout), lambda i: (0, 0, 0)),
        ],
        out_specs=pl.BlockSpec((nl, vb, rout), lambda i: (0, i, 0)),
        out_shape=jax.ShapeDtypeStruct((nl, V, rout), jnp.float32),
    )(h, bd)


def _tc_pool_fanout(p3, w, bd):
    nk, V, rin = p3.shape
    nl, _, rout = bd.shape
    vb = 256

    def body(p_ref, w_ref, bd_ref, y_ref):
        acc = p_ref[0] * w_ref[:, 0:1]
        for k in range(1, nk):
            acc = acc + p_ref[k] * w_ref[:, k:k + 1]
        hb = acc.astype(jnp.bfloat16)
        for l in range(nl):
            y_ref[l] = lax.dot(hb, bd_ref[l], preferred_element_type=jnp.float32)

    return pl.pallas_call(
        body,
        grid=(V // vb,),
        in_specs=[
            pl.BlockSpec((nk, vb, rin), lambda i: (0, i, 0)),
            pl.BlockSpec((vb, nk), lambda i: (i, 0)),
            pl.BlockSpec((nl, rin, rout), lambda i: (0, 0, 0)),
        ],
        out_specs=pl.BlockSpec((nl, vb, rout), lambda i: (0, i, 0)),
        out_shape=jax.ShapeDtypeStruct((nl, V, rout), jnp.float32),
    )(p3, w, bd)


def _tc_lsum(g3, bias_row, act):
    nl, V, R = g3.shape
    vb = 512

    def body(g_ref, b_ref, o_ref):
        acc = g_ref[0]
        for l in range(1, nl):
            acc = acc + g_ref[l]
        acc = acc + b_ref[...]
        if act:
            acc = jnp.where(acc > 0, acc, jnp.exp(acc) - 1.0)
        o_ref[...] = acc

    return pl.pallas_call(
        body,
        grid=(V // vb,),
        in_specs=[
            pl.BlockSpec((nl, vb, R), lambda i: (0, i, 0)),
            pl.BlockSpec((1, R), lambda i: (0, 0)),
        ],
        out_specs=pl.BlockSpec((vb, R), lambda i: (i, 0)),
        out_shape=jax.ShapeDtypeStruct((V, R), jnp.float32),
    )(g3, bias_row)


def _tc_pool(p3, w):
    nk, V, R = p3.shape
    vb = 512

    def body(p_ref, w_ref, o_ref):
        acc = p_ref[0] * w_ref[:, 0:1]
        for k in range(1, nk):
            acc = acc + p_ref[k] * w_ref[:, k:k + 1]
        o_ref[...] = acc

    return pl.pallas_call(
        body,
        grid=(V // vb,),
        in_specs=[
            pl.BlockSpec((nk, vb, R), lambda i: (0, i, 0)),
            pl.BlockSpec((vb, nk), lambda i: (i, 0)),
        ],
        out_specs=pl.BlockSpec((vb, R), lambda i: (i, 0)),
        out_shape=jax.ShapeDtypeStruct((V, R), jnp.float32),
    )(p3, w)


def _tc_latent(a, wen_t, ben_row, wde_t, bde_row):
    def body(a_ref, we_ref, be_ref, wd_ref, bd_ref, z_ref, d_ref):
        z = lax.dot(a_ref[...].astype(jnp.bfloat16), we_ref[...],
                    preferred_element_type=jnp.float32) + be_ref[...]
        z_ref[...] = z
        d_ref[...] = lax.dot(z.astype(jnp.bfloat16), wd_ref[...],
                             preferred_element_type=jnp.float32) + bd_ref[...]

    return pl.pallas_call(
        body,
        out_shape=(jax.ShapeDtypeStruct((a.shape[0], wen_t.shape[1]), jnp.float32),
                   jax.ShapeDtypeStruct((a.shape[0], wde_t.shape[1]), jnp.float32)),
    )(a, wen_t, ben_row, wde_t, bde_row)


def _flat(ix):
    return ix.astype(jnp.int32).T.reshape(-1)


def _flat_off(ix, n):
    f = ix.astype(jnp.int32).T
    return (f + jnp.arange(_L, dtype=jnp.int32)[:, None] * n).reshape(-1)


def _block_diag(W):
    co = W.shape[0]
    ci = W.shape[1] // _L
    wt = W.reshape(co, _L, ci).transpose(1, 2, 0)
    eye = jnp.eye(_B, dtype=W.dtype)
    bd = eye[None, :, None, :, None] * wt[:, None, :, None, :]
    return bd.reshape(_L, _B * ci, _B * co).astype(jnp.bfloat16)


def kernel(x, W_en, b_en, W_lin_en, b_lin_en, W_lin_de, b_lin_de, W_de, b_de,
           w_d, w_u, spiral_idx, idx_d, idx_u):
    v0, c0 = x.shape[1], x.shape[2]
    h = x.transpose(1, 0, 2).reshape(v0, _B * c0)

    for i in range(4):
        vi = spiral_idx[i].shape[0]
        g = _sc_gather(h, _flat(spiral_idx[i]))
        a = _tc_spiral_conv(g.reshape(_L, vi, -1), _block_diag(W_en[i]),
                            jnp.tile(b_en[i], _B)[None, :], act=True)
        vn = idx_d[i].shape[0]
        p = _sc_gather(a, _flat(idx_d[i]))
        h = _tc_pool(p.reshape(4, vn, -1), w_d[i])

    nv = h.shape[0]
    cl = h.shape[1] // _B
    perm1 = (jnp.arange(_B, dtype=jnp.int32)[:, None]
             + jnp.arange(nv, dtype=jnp.int32)[None, :] * _B).reshape(-1)
    a_lat = _sc_gather(h.reshape(nv * _B, cl), perm1).reshape(_B, nv * cl)
    z, dflat = _tc_latent(a_lat,
                          W_lin_en.T.astype(jnp.bfloat16), b_lin_en[None, :],
                          W_lin_de.T.astype(jnp.bfloat16), b_lin_de[None, :])
    perm2 = (jnp.arange(nv, dtype=jnp.int32)[:, None]
             + jnp.arange(_B, dtype=jnp.int32)[None, :] * nv).reshape(-1)
    d = _sc_gather(dflat.reshape(_B * nv, cl), perm2).reshape(nv, _B * cl)

    for j in range(4):
        lev = 3 - j
        vt = idx_u[lev].shape[0]
        p = _sc_gather(d, _flat(idx_u[lev]))
        bias_row = jnp.tile(b_de[j], _B)[None, :]
        if j in (0, 3):
            y = _tc_pool_fanout(p.reshape(3, vt, -1), w_u[lev],
                                _block_diag(W_de[j]))
            g = _sc_gather(y.reshape(_L * vt, -1), _flat_off(spiral_idx[lev], vt))
            d = _tc_lsum(g.reshape(_L, vt, -1), bias_row, act=(j < 3))
        else:
            d = _tc_pool(p.reshape(3, vt, -1), w_u[lev])
            g = _sc_gather(d, _flat(spiral_idx[lev]))
            d = _tc_spiral_conv(g.reshape(_L, vt, -1), _block_diag(W_de[j]),
                                bias_row, act=(j < 3))

    dout = d.reshape(v0, _B, -1).transpose(1, 0, 2)
    return (z, dout)


# speedup vs baseline: 2.2218x; 1.0907x over previous
import functools

import jax
import jax.numpy as jnp
from jax import lax
from jax.experimental import pallas as pl
from jax.experimental.pallas import tpu as pltpu
from jax.experimental.pallas import tpu_sc as plsc

_B = 32
_L = 9
_NW = 32
_CH = 64


def _sc_gather(table, idx):
    M = idx.shape[0]
    R = table.shape[1]
    per_w = M // _NW
    esz = table.dtype.itemsize
    ch = next(c for c in (128, 96, 64, 32, 16, 8)
              if per_w % (2 * c) == 0 and 2 * c * R * esz <= 400_000)
    nch = per_w // ch
    idx2 = idx.reshape(_NW, nch, ch)
    mesh = plsc.VectorSubcoreMesh(core_axis_name="c", subcore_axis_name="s")

    @functools.partial(
        pl.kernel,
        out_type=jax.ShapeDtypeStruct((M, R), table.dtype),
        mesh=mesh,
        compiler_params=pltpu.CompilerParams(
            use_tc_tiling_on_sc=(R * esz) % 512 == 0),
        scratch_types=[
            pltpu.VMEM((nch, ch), jnp.int32),
            pltpu.VMEM((ch, R), table.dtype),
            pltpu.VMEM((ch, R), table.dtype),
            pltpu.SemaphoreType.DMA,
            pltpu.SemaphoreType.DMA,
            pltpu.SemaphoreType.DMA,
            pltpu.SemaphoreType.DMA,
        ],
    )
    def k(tab_hbm, idx_hbm, out_hbm, idx_v, rows0, rows1, g0, g1, o0, o1):
        wid = lax.axis_index("s") * 2 + lax.axis_index("c")
        base = wid * nch
        bufs = ((rows0, g0, o0), (rows1, g1, o1))
        pltpu.sync_copy(idx_hbm.at[wid], idx_v)

        def orow(j):
            return pl.multiple_of((base + j) * ch, ch)
        for b, (rows, g, _) in enumerate(bufs):
            pltpu.async_copy(tab_hbm.at[idx_v.at[b]], rows, g)

        @pl.loop(0, nch - 2, step=2)
        def _(i):
            for b, (rows, g, o) in enumerate(bufs):
                pltpu.make_async_copy(tab_hbm.at[idx_v.at[i + b]], rows, g).wait()
                pltpu.async_copy(rows, out_hbm.at[pl.ds(orow(i + b), ch)], o)
            for b, (rows, g, o) in enumerate(bufs):
                pltpu.make_async_copy(
                    rows, out_hbm.at[pl.ds(orow(i + b), ch)], o).wait()
                pltpu.async_copy(tab_hbm.at[idx_v.at[i + 2 + b]], rows, g)

        for b, (rows, g, o) in enumerate(bufs):
            j = nch - 2 + b
            pltpu.make_async_copy(tab_hbm.at[idx_v.at[j]], rows, g).wait()
            pltpu.async_copy(rows, out_hbm.at[pl.ds(orow(j), ch)], o)
        for b, (rows, g, o) in enumerate(bufs):
            j = nch - 2 + b
            pltpu.make_async_copy(
                rows, out_hbm.at[pl.ds(orow(j), ch)], o).wait()

    return k(table, idx2)


def _tc_spiral_conv(g3, bd, bias_row, act):
    nl, V, rin = g3.shape
    rout = bd.shape[2]
    vb = 512 if max(rin, rout) <= 512 else 256

    def body(g_ref, w_ref, b_ref, o_ref):
        acc = jnp.zeros((vb, rout), jnp.float32)
        for l in range(nl):
            acc = acc + lax.dot(
                g_ref[l].astype(jnp.bfloat16), w_ref[l],
                preferred_element_type=jnp.float32)
        acc = acc + b_ref[...]
        if act:
            acc = jnp.where(acc > 0, acc, jnp.exp(acc) - 1.0)
        o_ref[...] = acc

    return pl.pallas_call(
        body,
        grid=(V // vb,),
        in_specs=[
            pl.BlockSpec((nl, vb, rin), lambda i: (0, i, 0)),
            pl.BlockSpec((nl, rin, rout), lambda i: (0, 0, 0)),
            pl.BlockSpec((1, rout), lambda i: (0, 0)),
        ],
        out_specs=pl.BlockSpec((vb, rout), lambda i: (i, 0)),
        out_shape=jax.ShapeDtypeStruct((V, rout), jnp.float32),
    )(g3, bd, bias_row)


def _tc_fanout(h, bd):
    V, rin = h.shape
    nl, _, rout = bd.shape
    vb = 512 if max(rin, rout) <= 512 else 256

    def body(h_ref, w_ref, y_ref):
        hb = h_ref[...].astype(jnp.bfloat16)
        for l in range(nl):
            y_ref[l] = lax.dot(hb, w_ref[l], preferred_element_type=jnp.float32)

    return pl.pallas_call(
        body,
        grid=(V // vb,),
        in_specs=[
            pl.BlockSpec((vb, rin), lambda i: (i, 0)),
            pl.BlockSpec((nl, rin, rout), lambda i: (0, 0, 0)),
        ],
        out_specs=pl.BlockSpec((nl, vb, rout), lambda i: (0, i, 0)),
        out_shape=jax.ShapeDtypeStruct((nl, V, rout), jnp.float32),
    )(h, bd)


def _tc_pool_fanout(p3, w, bd):
    nk, V, rin = p3.shape
    nl, _, rout = bd.shape
    vb = 256

    def body(p_ref, w_ref, bd_ref, y_ref):
        acc = p_ref[0] * w_ref[:, 0:1]
        for k in range(1, nk):
            acc = acc + p_ref[k] * w_ref[:, k:k + 1]
        hb = acc.astype(jnp.bfloat16)
        for l in range(nl):
            y_ref[l] = lax.dot(hb, bd_ref[l], preferred_element_type=jnp.float32)

    return pl.pallas_call(
        body,
        grid=(V // vb,),
        in_specs=[
            pl.BlockSpec((nk, vb, rin), lambda i: (0, i, 0)),
            pl.BlockSpec((vb, nk), lambda i: (i, 0)),
            pl.BlockSpec((nl, rin, rout), lambda i: (0, 0, 0)),
        ],
        out_specs=pl.BlockSpec((nl, vb, rout), lambda i: (0, i, 0)),
        out_shape=jax.ShapeDtypeStruct((nl, V, rout), jnp.float32),
    )(p3, w, bd)


def _tc_lsum(g3, bias_row, act, rout=None):
    nl, V, R = g3.shape
    rout = R if rout is None else rout
    vb = 512

    def body(g_ref, b_ref, o_ref):
        acc = g_ref[0]
        for l in range(1, nl):
            acc = acc + g_ref[l]
        acc = acc[:, :rout] + b_ref[...]
        if act:
            acc = jnp.where(acc > 0, acc, jnp.exp(acc) - 1.0)
        o_ref[...] = acc

    return pl.pallas_call(
        body,
        grid=(V // vb,),
        in_specs=[
            pl.BlockSpec((nl, vb, R), lambda i: (0, i, 0)),
            pl.BlockSpec((1, rout), lambda i: (0, 0)),
        ],
        out_specs=pl.BlockSpec((vb, rout), lambda i: (i, 0)),
        out_shape=jax.ShapeDtypeStruct((V, rout), jnp.float32),
    )(g3, bias_row)


def _tc_pool(p3, w):
    nk, V, R = p3.shape
    vb = 512

    def body(p_ref, w_ref, o_ref):
        acc = p_ref[0] * w_ref[:, 0:1]
        for k in range(1, nk):
            acc = acc + p_ref[k] * w_ref[:, k:k + 1]
        o_ref[...] = acc

    return pl.pallas_call(
        body,
        grid=(V // vb,),
        in_specs=[
            pl.BlockSpec((nk, vb, R), lambda i: (0, i, 0)),
            pl.BlockSpec((vb, nk), lambda i: (i, 0)),
        ],
        out_specs=pl.BlockSpec((vb, R), lambda i: (i, 0)),
        out_shape=jax.ShapeDtypeStruct((V, R), jnp.float32),
    )(p3, w)


def _tc_latent(a, wen_t, ben_row, wde_t, bde_row):
    def body(a_ref, we_ref, be_ref, wd_ref, bd_ref, z_ref, d_ref):
        z = lax.dot(a_ref[...].astype(jnp.bfloat16), we_ref[...],
                    preferred_element_type=jnp.float32) + be_ref[...]
        z_ref[...] = z
        d_ref[...] = lax.dot(z.astype(jnp.bfloat16), wd_ref[...],
                             preferred_element_type=jnp.float32) + bd_ref[...]

    return pl.pallas_call(
        body,
        out_shape=(jax.ShapeDtypeStruct((a.shape[0], wen_t.shape[1]), jnp.float32),
                   jax.ShapeDtypeStruct((a.shape[0], wde_t.shape[1]), jnp.float32)),
    )(a, wen_t, ben_row, wde_t, bde_row)


def _flat(ix):
    return ix.astype(jnp.int32).T.reshape(-1)


def _flat_off(ix, n):
    f = ix.astype(jnp.int32).T
    return (f + jnp.arange(_L, dtype=jnp.int32)[:, None] * n).reshape(-1)


def _block_diag(W, pad_in=0, pad_out=0):
    co = W.shape[0]
    ci = W.shape[1] // _L
    wt = W.reshape(co, _L, ci).transpose(1, 2, 0)
    eye = jnp.eye(_B, dtype=W.dtype)
    bd = (eye[None, :, None, :, None] * wt[:, None, :, None, :]
          ).reshape(_L, _B * ci, _B * co)
    bd = jnp.pad(bd, ((0, 0), (0, pad_in), (0, pad_out)))
    return bd.astype(jnp.bfloat16)


def kernel(x, W_en, b_en, W_lin_en, b_lin_en, W_lin_de, b_lin_de, W_de, b_de,
           w_d, w_u, spiral_idx, idx_d, idx_u):
    v0, c0 = x.shape[1], x.shape[2]
    h = jnp.pad(x.transpose(1, 0, 2).reshape(v0, _B * c0), ((0, 0), (0, 32)))

    for i in range(4):
        vi = spiral_idx[i].shape[0]
        g = _sc_gather(h, _flat(spiral_idx[i]))
        a = _tc_spiral_conv(g.reshape(_L, vi, -1),
                            _block_diag(W_en[i], pad_in=32 if i == 0 else 0),
                            jnp.tile(b_en[i], _B)[None, :], act=True)
        vn = idx_d[i].shape[0]
        p = _sc_gather(a, _flat(idx_d[i]))
        h = _tc_pool(p.reshape(4, vn, -1), w_d[i])

    nv = h.shape[0]
    cl = h.shape[1] // _B
    perm1 = (jnp.arange(_B, dtype=jnp.int32)[:, None]
             + jnp.arange(nv, dtype=jnp.int32)[None, :] * _B).reshape(-1)
    a_lat = _sc_gather(h.reshape(nv * _B, cl), perm1).reshape(_B, nv * cl)
    z, dflat = _tc_latent(a_lat,
                          W_lin_en.T.astype(jnp.bfloat16), b_lin_en[None, :],
                          W_lin_de.T.astype(jnp.bfloat16), b_lin_de[None, :])
    perm2 = (jnp.arange(nv, dtype=jnp.int32)[:, None]
             + jnp.arange(_B, dtype=jnp.int32)[None, :] * nv).reshape(-1)
    d = _sc_gather(dflat.reshape(_B * nv, cl), perm2).reshape(nv, _B * cl)

    for j in range(4):
        lev = 3 - j
        vt = idx_u[lev].shape[0]
        p = _sc_gather(d, _flat(idx_u[lev]))
        bias_row = jnp.tile(b_de[j], _B)[None, :]
        if j in (0, 3):
            y = _tc_pool_fanout(p.reshape(3, vt, -1), w_u[lev],
                                _block_diag(W_de[j], pad_out=32 if j == 3 else 0))
            g = _sc_gather(y.reshape(_L * vt, -1), _flat_off(spiral_idx[lev], vt))
            d = _tc_lsum(g.reshape(_L, vt, -1), bias_row, act=(j < 3),
                         rout=_B * 3 if j == 3 else None)
        else:
            d = _tc_pool(p.reshape(3, vt, -1), w_u[lev])
            g = _sc_gather(d, _flat(spiral_idx[lev]))
            d = _tc_spiral_conv(g.reshape(_L, vt, -1), _block_diag(W_de[j]),
                                bias_row, act=(j < 3))

    dout = d.reshape(v0, _B, -1).transpose(1, 0, 2)
    return (z, dout)
